# baseline (speedup 1.0000x reference)
import functools

import jax
import jax.numpy as jnp
from jax import lax
from jax.experimental import pallas as pl
from jax.experimental.pallas import tpu as pltpu
from jax.experimental.pallas import tpu_sc as plsc

N = 10000
E = 320000
NC = 2
NS = 16
NW = NC * NS
E_PER_W = E // NW
CHUNK = 2000
N_CHUNKS = E_PER_W // CHUNK
GROUPS = CHUNK // 16
WIDE = 5

BOHR2 = 0.529177 ** 2
KE_KCAL = 332.0637
R_ON = 5.0 / 4.0
INV_RANGE = 1.0 / (3.0 * 5.0 / 4.0 - 5.0 / 4.0)


def _rsqrt16(v):
    bits = plsc.bitcast(v, jnp.int32)
    y = plsc.bitcast(jnp.full((16,), 0x5F3759DF, jnp.int32) - (bits >> 1),
                     jnp.float32)
    y = y * (1.5 - 0.5 * v * y * y)
    y = y * (1.5 - 0.5 * v * y * y)
    return y


def _sc_edge_body(x_hbm, y_hbm, z_hbm, tc_hbm, nbrs_hbm, offs_hbm, out_hbm,
                  x_v, y_v, z_v, tc_v,
                  iv_a, jv_a, ox_a, oy_a, oz_a,
                  iv_b, jv_b, ox_b, oy_b, oz_b, acc_v,
                  sem_t, sem_a, sem_b):
    wid = lax.axis_index("s") * NC + lax.axis_index("c")
    ebase = wid * E_PER_W

    t0 = pltpu.async_copy(x_hbm, x_v, sem_t)
    t1 = pltpu.async_copy(y_hbm, y_v, sem_t)
    t2 = pltpu.async_copy(z_hbm, z_v, sem_t)
    t3 = pltpu.async_copy(tc_hbm, tc_v, sem_t)

    bufs = ((iv_a, jv_a, ox_a, oy_a, oz_a, sem_a),
            (iv_b, jv_b, ox_b, oy_b, oz_b, sem_b))
    hbms = ((nbrs_hbm, 0), (nbrs_hbm, E), (offs_hbm, 0), (offs_hbm, E),
            (offs_hbm, 2 * E))

    def _fire(c, which):
        cb = ebase + c * CHUNK
        for (h, off), d in zip(hbms, bufs[which][:5]):
            pltpu.async_copy(h.at[pl.ds(off + cb, CHUNK)], d, bufs[which][5])

    def _drain(c, which):
        cb = ebase + c * CHUNK
        for (h, off), d in zip(hbms, bufs[which][:5]):
            pltpu.make_async_copy(h.at[pl.ds(off + cb, CHUNK)], d,
                                  bufs[which][5]).wait()

    _fire(0, 0)

    zeros = jnp.zeros((16,), jnp.float32)

    def _zero(g, carry):
        acc_v[pl.ds(g * 16, 16)] = zeros
        return carry

    lax.fori_loop(0, N // 16, _zero, 0, unroll=4)

    t0.wait()
    t1.wait()
    t2.wait()
    t3.wait()

    def _edge_groups(which):
        iv_v, jv_v, ox_v, oy_v, oz_v, _ = bufs[which]

        def _edge_block(b, carry):
            cols = []
            for k in range(WIDE):
                sl = pl.ds((b * WIDE + k) * 16, 16)
                iv = iv_v[sl]
                jv = jv_v[sl]
                cols.append((
                    iv, jv,
                    plsc.load_gather(x_v, [iv]) - plsc.load_gather(x_v, [jv])
                    - ox_v[sl],
                    plsc.load_gather(y_v, [iv]) - plsc.load_gather(y_v, [jv])
                    - oy_v[sl],
                    plsc.load_gather(z_v, [iv]) - plsc.load_gather(z_v, [jv])
                    - oz_v[sl],
                    plsc.load_gather(tc_v, [iv]) * plsc.load_gather(tc_v, [jv]),
                ))
            outs = []
            for iv, jv, rx, ry, rz, qq in cols:
                d2 = rx * rx + ry * ry + rz * rz
                d2m = jnp.maximum(d2, 1e-12)
                inv_d = _rsqrt16(d2m)
                dist = d2m * inv_d
                x = jnp.clip((dist - R_ON) * INV_RANGE, 0.0, 1.0)
                x3 = x * x * x
                fs = 1.0 - x3 * (10.0 + x * (-15.0 + 6.0 * x))
                arg0 = fs * _rsqrt16(d2 + BOHR2)
                arg1 = (1.0 - fs) * inv_d
                p = KE_KCAL * qq * (arg0 + arg1)
                outs.append((iv, jnp.where(jv > iv, p, 0.0)))
            for iv, p in outs:
                plsc.addupdate_scatter(acc_v, [iv], p)
            return carry

        lax.fori_loop(0, GROUPS // WIDE, _edge_block, 0)

    def _pair(c, _):
        c2 = c + c
        _drain(c2, 0)
        _fire(c2 + 1, 1)
        _edge_groups(0)
        _drain(c2 + 1, 1)
        _fire(c2 + 2, 0)
        _edge_groups(1)
        return _

    lax.fori_loop(0, (N_CHUNKS - 1) // 2, _pair, 0)
    _drain(N_CHUNKS - 1, 0)
    _edge_groups(0)

    pltpu.sync_copy(acc_v, out_hbm.at[wid])


@jax.jit
def _sc_edge_energy(x, y, z, total_charge, nbrs_t, offs_t):
    mesh = plsc.VectorSubcoreMesh(core_axis_name="c", subcore_axis_name="s",
                                  num_cores=NC, num_subcores=NS)
    return pl.kernel(
        _sc_edge_body,
        out_type=jax.ShapeDtypeStruct((NW, N), jnp.float32),
        mesh=mesh,
        compiler_params=pltpu.CompilerParams(needs_layout_passes=False,
                                             use_tc_tiling_on_sc=False),
        scratch_types=[
            pltpu.VMEM((N,), jnp.float32),
            pltpu.VMEM((N,), jnp.float32),
            pltpu.VMEM((N,), jnp.float32),
            pltpu.VMEM((N,), jnp.float32),
            pltpu.VMEM((CHUNK,), jnp.int32),
            pltpu.VMEM((CHUNK,), jnp.int32),
            pltpu.VMEM((CHUNK,), jnp.float32),
            pltpu.VMEM((CHUNK,), jnp.float32),
            pltpu.VMEM((CHUNK,), jnp.float32),
            pltpu.VMEM((CHUNK,), jnp.int32),
            pltpu.VMEM((CHUNK,), jnp.int32),
            pltpu.VMEM((CHUNK,), jnp.float32),
            pltpu.VMEM((CHUNK,), jnp.float32),
            pltpu.VMEM((CHUNK,), jnp.float32),
            pltpu.VMEM((N,), jnp.float32),
            pltpu.SemaphoreType.DMA,
            pltpu.SemaphoreType.DMA,
            pltpu.SemaphoreType.DMA,
        ],
    )(x, y, z, total_charge, nbrs_t, offs_t)


def _tc_finish_body(part_ref, tc_ref, xyzt_ref, e_ref, d_ref):
    e_ref[...] = jnp.sum(part_ref[...], axis=0)
    d_ref[...] = tc_ref[...] * xyzt_ref[...]


@jax.jit
def _tc_finish(partial, tc2, xyzt):
    return pl.pallas_call(
        _tc_finish_body,
        out_shape=(
            jax.ShapeDtypeStruct((N,), jnp.float32),
            jax.ShapeDtypeStruct((3, N), jnp.float32),
        ),
    )(partial, tc2, xyzt)


def kernel(s_i, v_i, z, xyz, total_charge, num_atoms, mol_nbrs, mol_offsets,
           W1, Wb0_w, Wb0_b, Wb1_w, Wb1_b):
    partial = _sc_edge_energy(
        xyz[:, 0], xyz[:, 1], xyz[:, 2], total_charge,
        mol_nbrs.T.reshape(-1), mol_offsets.T.reshape(-1))
    e1, dipt = _tc_finish(partial, total_charge.reshape(1, N), xyz.T)
    return (e1.reshape(N, 1), total_charge.reshape(N, 1), dipt.T)


# speedup vs baseline: 1.0317x; 1.0317x over previous
import functools

import jax
import jax.numpy as jnp
from jax import lax
from jax.experimental import pallas as pl
from jax.experimental.pallas import tpu as pltpu
from jax.experimental.pallas import tpu_sc as plsc

N = 10000
E = 320000
NC = 2
NS = 16
NW = NC * NS
E_PER_W = E // NW
CHUNK = 2000
N_CHUNKS = E_PER_W // CHUNK
GROUPS = CHUNK // 16
WIDE = 5

BOHR2 = 0.529177 ** 2
KE_KCAL = 332.0637
R_ON = 5.0 / 4.0
INV_RANGE = 1.0 / (3.0 * 5.0 / 4.0 - 5.0 / 4.0)


def _rsqrt16(v):
    bits = plsc.bitcast(v, jnp.int32)
    y = plsc.bitcast(jnp.full((16,), 0x5F3759DF, jnp.int32) - (bits >> 1),
                     jnp.float32)
    y = y * (1.5 - 0.5 * v * y * y)
    y = y * (1.5 - 0.5 * v * y * y)
    return y


def _sc_edge_body(x_hbm, y_hbm, z_hbm, tc_hbm, nbrs_hbm, offs_hbm, out_hbm,
                  x_v, y_v, z_v, tc_v,
                  iv_a, jv_a, ox_a, oy_a, oz_a,
                  iv_b, jv_b, ox_b, oy_b, oz_b, acc_v,
                  sem_t, sem_a, sem_b):
    wid = lax.axis_index("s") * NC + lax.axis_index("c")
    ebase = wid * E_PER_W

    t0 = pltpu.async_copy(x_hbm, x_v, sem_t)
    t1 = pltpu.async_copy(y_hbm, y_v, sem_t)
    t2 = pltpu.async_copy(z_hbm, z_v, sem_t)
    t3 = pltpu.async_copy(tc_hbm, tc_v, sem_t)

    bufs = ((iv_a, jv_a, ox_a, oy_a, oz_a, sem_a),
            (iv_b, jv_b, ox_b, oy_b, oz_b, sem_b))
    hbms = ((nbrs_hbm, 0), (nbrs_hbm, E), (offs_hbm, 0), (offs_hbm, E),
            (offs_hbm, 2 * E))

    def _fire(c, which):
        cb = ebase + c * CHUNK
        for (h, off), d in zip(hbms, bufs[which][:5]):
            pltpu.async_copy(h.at[pl.ds(off + cb, CHUNK)], d, bufs[which][5])

    def _drain(c, which):
        cb = ebase + c * CHUNK
        for (h, off), d in zip(hbms, bufs[which][:5]):
            pltpu.make_async_copy(h.at[pl.ds(off + cb, CHUNK)], d,
                                  bufs[which][5]).wait()

    _fire(0, 0)

    zeros = jnp.zeros((16,), jnp.float32)

    def _zero(g, carry):
        acc_v[pl.ds(g * 16, 16)] = zeros
        return carry

    lax.fori_loop(0, N // 16, _zero, 0, unroll=4)

    t0.wait()
    t1.wait()
    t2.wait()
    t3.wait()

    def _edge_groups(which):
        iv_v, jv_v, ox_v, oy_v, oz_v, _ = bufs[which]

        def _edge_block(b, carry):
            cols = []
            for k in range(WIDE):
                sl = pl.ds((b * WIDE + k) * 16, 16)
                iv = iv_v[sl]
                jv = jv_v[sl]
                cols.append((
                    iv, jv,
                    plsc.load_gather(x_v, [iv]) - plsc.load_gather(x_v, [jv])
                    - ox_v[sl],
                    plsc.load_gather(y_v, [iv]) - plsc.load_gather(y_v, [jv])
                    - oy_v[sl],
                    plsc.load_gather(z_v, [iv]) - plsc.load_gather(z_v, [jv])
                    - oz_v[sl],
                    plsc.load_gather(tc_v, [iv]) * plsc.load_gather(tc_v, [jv]),
                ))
            outs = []
            for iv, jv, rx, ry, rz, qq in cols:
                d2 = rx * rx + ry * ry + rz * rz
                d2m = jnp.maximum(d2, 1e-12)
                inv_d = _rsqrt16(d2m)
                dist = d2m * inv_d
                x = jnp.clip((dist - R_ON) * INV_RANGE, 0.0, 1.0)
                x3 = x * x * x
                fs = 1.0 - x3 * (10.0 + x * (-15.0 + 6.0 * x))
                arg0 = fs * _rsqrt16(d2 + BOHR2)
                arg1 = (1.0 - fs) * inv_d
                p = KE_KCAL * qq * (arg0 + arg1)
                outs.append((iv, jnp.where(jv > iv, p, 0.0)))
            for iv, p in outs:
                plsc.addupdate_scatter(acc_v, [iv], p)
            return carry

        lax.fori_loop(0, GROUPS // WIDE, _edge_block, 0)

    def _pair(c, _):
        c2 = c + c
        _drain(c2, 0)
        _fire(c2 + 1, 1)
        _edge_groups(0)
        _drain(c2 + 1, 1)
        _fire(c2 + 2, 0)
        _edge_groups(1)
        return _

    lax.fori_loop(0, (N_CHUNKS - 1) // 2, _pair, 0)
    _drain(N_CHUNKS - 1, 0)
    _edge_groups(0)

    pltpu.sync_copy(acc_v, out_hbm.at[wid])


@jax.jit
def _sc_edge_energy(x, y, z, total_charge, nbrs_t, offs_t):
    mesh = plsc.VectorSubcoreMesh(core_axis_name="c", subcore_axis_name="s",
                                  num_cores=NC, num_subcores=NS)
    return pl.kernel(
        _sc_edge_body,
        out_type=jax.ShapeDtypeStruct((NW, N), jnp.float32),
        mesh=mesh,
        compiler_params=pltpu.CompilerParams(needs_layout_passes=False,
                                             use_tc_tiling_on_sc=False),
        scratch_types=[
            pltpu.VMEM((N,), jnp.float32),
            pltpu.VMEM((N,), jnp.float32),
            pltpu.VMEM((N,), jnp.float32),
            pltpu.VMEM((N,), jnp.float32),
            pltpu.VMEM((CHUNK,), jnp.int32),
            pltpu.VMEM((CHUNK,), jnp.int32),
            pltpu.VMEM((CHUNK,), jnp.float32),
            pltpu.VMEM((CHUNK,), jnp.float32),
            pltpu.VMEM((CHUNK,), jnp.float32),
            pltpu.VMEM((CHUNK,), jnp.int32),
            pltpu.VMEM((CHUNK,), jnp.int32),
            pltpu.VMEM((CHUNK,), jnp.float32),
            pltpu.VMEM((CHUNK,), jnp.float32),
            pltpu.VMEM((CHUNK,), jnp.float32),
            pltpu.VMEM((N,), jnp.float32),
            pltpu.SemaphoreType.DMA,
            pltpu.SemaphoreType.DMA,
            pltpu.SemaphoreType.DMA,
        ],
    )(x, y, z, total_charge, nbrs_t, offs_t)


def _tc_finish_body(part_ref, tc_ref, xyzt_ref, e_ref, q_ref, d_ref):
    e_ref[...] = jnp.sum(part_ref[...], axis=0, keepdims=True)
    t = tc_ref[...]
    q_ref[...] = t
    d_ref[...] = t * xyzt_ref[...]


@jax.jit
def _tc_finish(partial, tc2, xyzt):
    return pl.pallas_call(
        _tc_finish_body,
        out_shape=(
            jax.ShapeDtypeStruct((1, N), jnp.float32),
            jax.ShapeDtypeStruct((1, N), jnp.float32),
            jax.ShapeDtypeStruct((3, N), jnp.float32),
        ),
    )(partial, tc2, xyzt)


def kernel(s_i, v_i, z, xyz, total_charge, num_atoms, mol_nbrs, mol_offsets,
           W1, Wb0_w, Wb0_b, Wb1_w, Wb1_b):
    partial = _sc_edge_energy(
        xyz[:, 0], xyz[:, 1], xyz[:, 2], total_charge,
        mol_nbrs.T.reshape(-1), mol_offsets.T.reshape(-1))
    e2, q2, dipt = _tc_finish(partial, total_charge.reshape(1, N), xyz.T)
    return (e2.reshape(N, 1), q2.reshape(N, 1), dipt.T)
